# baseline (speedup 1.0000x reference)
import functools

import jax
import jax.numpy as jnp
from jax import lax
from jax.experimental import pallas as pl
from jax.experimental.pallas import tpu as pltpu
from jax.experimental.pallas import tpu_sc as plsc

DIM = 64
LANES = 16
IDX_ROW = 128
T_SUB = 256
T_SUPER = 512


def _make_kernel(num_tokens, vocab):
    info = plsc.get_sparse_core_info()
    num_workers = info.num_cores * info.num_subcores
    per_w = num_tokens // num_workers
    assert per_w * num_workers == num_tokens
    assert per_w % T_SUPER == 0
    n_super = per_w // T_SUPER
    n_sub = T_SUPER // T_SUB
    n_streams = (2 * T_SUB) // IDX_ROW
    idx_rows_super = (2 * T_SUPER) // IDX_ROW

    mesh = plsc.VectorSubcoreMesh(core_axis_name="c", subcore_axis_name="s")

    @functools.partial(
        pl.kernel,
        mesh=mesh,
        out_type=jax.ShapeDtypeStruct((num_tokens // 2, 2 * DIM), jnp.float32),
        scratch_types=[
            pltpu.VMEM((idx_rows_super, IDX_ROW), jnp.int32),
            pltpu.VMEM((2 * T_SUB, 2 * DIM), jnp.float32),
            pltpu.VMEM((T_SUB // 2, 2 * DIM), jnp.float32),
            pltpu.SemaphoreType.DMA,
        ],
    )
    def k(idx_hbm, table_hbm, out_hbm, idx_v, rows_v, out_v, sem):
        wid = lax.axis_index("s") * info.num_cores + lax.axis_index("c")
        tok0 = wid * per_w

        def super_body(g, carry):
            base = pl.multiple_of(tok0 + g * T_SUPER, T_SUPER)
            idx_row0 = pl.multiple_of((base * 2) // IDX_ROW, 8)
            pltpu.sync_copy(idx_hbm.at[pl.ds(idx_row0, idx_rows_super)],
                            idx_v)
            for sub in range(n_sub):
                copies = [
                    pltpu.async_copy(
                        table_hbm.at[idx_v.at[sub * n_streams + j]],
                        rows_v.at[pl.ds(j * IDX_ROW, IDX_ROW)],
                        sem,
                    )
                    for j in range(n_streams)
                ]
                for c in copies:
                    c.wait()

                def pair_body(u, c2):
                    for half in range(2):
                        for kk in range(DIM // LANES):
                            s_in = pl.ds(kk * LANES, LANES)
                            s_out = pl.ds(half * DIM + kk * LANES, LANES)
                            r0 = 4 * u + 2 * half
                            out_v[u, s_out] = (rows_v[r0, s_in]
                                               + rows_v[r0 + 1, s_in])
                    return c2

                lax.fori_loop(0, T_SUB // 2, pair_body, 0, unroll=2)
                pair0 = pl.multiple_of((base + sub * T_SUB) // 2, T_SUB // 2)
                pltpu.sync_copy(out_v, out_hbm.at[pl.ds(pair0, T_SUB // 2)])
            return carry

        lax.fori_loop(0, n_super, super_body, 0)

    return k


def kernel(batch_pos_list, table):
    B, L, P = batch_pos_list.shape
    assert P == 2
    V, D = table.shape
    assert D == DIM
    N = B * L
    idx2d = batch_pos_list.reshape(N * P // IDX_ROW, IDX_ROW)
    table_p = jnp.pad(table, ((0, 0), (0, 2 * DIM - D)))
    k = _make_kernel(N, V)
    out = k(idx2d, table_p)
    return out.reshape(B, L, D)
